# baseline (speedup 1.0000x reference)
import jax
import jax.numpy as jnp
from jax import lax
from jax.experimental import pallas as pl
from jax.experimental.pallas import tpu as pltpu
from jax.experimental.pallas import tpu_sc as plsc

_B = 16384
_L = 16
_NC = 2
_NS = 16
_NW = _NC * _NS
_BPW = _B // _NW
_C = 64
_NCH = _BPW // _C
_G = _C // _L
_MARGIN = 1.0
_UNROLL = 4
_D = 64
_W = 128


def _transe_body(xT, ent2, rel2, out, idxb, gidxb, colb, r0, r1, obuf,
                 sem_x, sem0, sem1):
    wid = lax.axis_index("s") * _NC + lax.axis_index("c")
    base = wid * _BPW

    cps = [pltpu.async_copy(xT.at[j, pl.ds(base, _BPW)], idxb.at[j], sem_x)
           for j in range(6)]
    for cp in cps:
        cp.wait()

    iota = lax.iota(jnp.int32, _L)

    def tbody(g, _):
        sl = pl.ds(g * _L, _L)
        for j in range(6):
            v = idxb[j, sl]
            gidxb[j, sl] = lax.shift_right_logical(v, 1)
            colb[j, sl] = lax.shift_left(v & 1, 6)
        return 0

    lax.fori_loop(0, _BPW // _L, tbody, 0)

    rowbufs = (r0, r1)
    sems = (sem0, sem1)

    def fire(k, s):
        sl = pl.ds(k * _C, _C)
        rb, sm = rowbufs[s], sems[s]
        return [
            pltpu.async_copy(ent2.at[gidxb.at[0, sl]], rb.at[0], sm),
            pltpu.async_copy(ent2.at[gidxb.at[1, sl]], rb.at[1], sm),
            pltpu.async_copy(rel2.at[gidxb.at[2, sl]], rb.at[2], sm),
            pltpu.async_copy(ent2.at[gidxb.at[3, sl]], rb.at[3], sm),
            pltpu.async_copy(ent2.at[gidxb.at[4, sl]], rb.at[4], sm),
            pltpu.async_copy(rel2.at[gidxb.at[5, sl]], rb.at[5], sm),
        ]

    acc = jnp.zeros((_L,), jnp.float32)
    cps = fire(0, 0)
    for k in range(_NCH):
        nxt = fire(k + 1, (k + 1) % 2) if k + 1 < _NCH else None
        for cp in cps:
            cp.wait()
        rb = rowbufs[k % 2]

        def gbody(g, acc, rb=rb, k=k):
            rows = g * _L + iota
            gsl = pl.ds(k * _C + g * _L, _L)
            cb = [colb[j, gsl] for j in range(6)]

            def cbody(ci, rowsum, rb=rb, cb=cb, rows=rows):
                for u in range(_UNROLL):
                    c = ci * _UNROLL + u
                    a = plsc.load_gather(rb.at[0], [rows, cb[0] + c])
                    t = plsc.load_gather(rb.at[1], [rows, cb[1] + c])
                    b = plsc.load_gather(rb.at[2], [rows, cb[2] + c])
                    d = plsc.load_gather(rb.at[3], [rows, cb[3] + c])
                    e = plsc.load_gather(rb.at[4], [rows, cb[4] + c])
                    f = plsc.load_gather(rb.at[5], [rows, cb[5] + c])
                    rowsum = rowsum + (jnp.abs(a + b - t) - jnp.abs(d + f - e))
                return rowsum

            rowsum = lax.fori_loop(0, _D // _UNROLL, cbody,
                                   jnp.zeros((_L,), jnp.float32))
            return acc + jnp.maximum(rowsum + _MARGIN, 0.0)

        acc = lax.fori_loop(0, _G, gbody, acc)
        cps = nxt

    obuf[...] = acc
    pltpu.sync_copy(obuf, out.at[wid])


def _transe_partials(xT, ent2, rel2):
    f32 = jnp.float32
    run = pl.kernel(
        _transe_body,
        mesh=plsc.VectorSubcoreMesh(core_axis_name="c", subcore_axis_name="s"),
        compiler_params=pltpu.CompilerParams(needs_layout_passes=False),
        out_type=jax.ShapeDtypeStruct((_NW, _L), f32),
        scratch_types=[
            pltpu.VMEM((6, _BPW), jnp.int32),
            pltpu.VMEM((6, _BPW), jnp.int32),
            pltpu.VMEM((6, _BPW), jnp.int32),
            pltpu.VMEM((6, _C, _W), f32),
            pltpu.VMEM((6, _C, _W), f32),
            pltpu.VMEM((_L,), f32),
            pltpu.SemaphoreType.DMA,
            pltpu.SemaphoreType.DMA,
            pltpu.SemaphoreType.DMA,
        ],
    )
    return run(xT, ent2, rel2)


def kernel(x, ent_emb, rel_emb):
    xT = x.T
    ent2 = ent_emb.reshape(500000, _W)
    rel2 = rel_emb.reshape(500, _W)
    partials = _transe_partials(xT, ent2, rel2)
    return jnp.sum(partials)


# speedup vs baseline: 1.5580x; 1.5580x over previous
import jax
import jax.numpy as jnp
from jax import lax
from jax.experimental import pallas as pl
from jax.experimental.pallas import tpu as pltpu
from jax.experimental.pallas import tpu_sc as plsc

_B = 16384
_L = 16
_NC = 2
_NS = 16
_NW = _NC * _NS
_BPW = _B // _NW
_C = 64
_NCH = _BPW // _C
_G = _C // _L
_MARGIN = 1.0
_UNROLL = 4
_D = 64
_W = 128
_ER = 4 * _C


def _transe_body(xT, ent, rel2, out, idxb, gidxb, colb, e0, e1, r0, r1,
                 obuf, sem_x, sem0, sem1):
    wid = lax.axis_index("s") * _NC + lax.axis_index("c")
    base = wid * _BPW

    cps = [pltpu.async_copy(xT.at[j, pl.ds(base, _BPW)], idxb.at[j], sem_x)
           for j in range(6)]
    for cp in cps:
        cp.wait()

    iota = lax.iota(jnp.int32, _L)

    def tbody(g, _):
        sl = pl.ds(g * _L, _L)
        for j, src in ((0, 2), (1, 5)):
            v = idxb[src, sl]
            gidxb[j, sl] = lax.shift_right_logical(v, 1)
            colb[j, sl] = lax.shift_left(v & 1, 6)
        return 0

    lax.fori_loop(0, _BPW // _L, tbody, 0)

    ebufs = (e0, e1)
    rbufs = (r0, r1)
    sems = (sem0, sem1)

    def fire(k, s):
        sl = pl.ds(k * _C, _C)
        eb, rb, sm = ebufs[s], rbufs[s], sems[s]
        rel_cps = [
            pltpu.async_copy(rel2.at[gidxb.at[0, sl]], rb.at[0], sm),
            pltpu.async_copy(rel2.at[gidxb.at[1, sl]], rb.at[1], sm),
        ]

        def issue(g, _, eb=eb, sm=sm, k=k):
            for j, role in enumerate((0, 1, 3, 4)):
                v = idxb[role, pl.ds(k * _C + g * _L, _L)]
                rowbase = j * _C + g * _L
                for lane in range(_L):
                    e = v[lane]
                    pltpu.async_copy(ent.at[pl.ds(e, 1)],
                                     eb.at[pl.ds(rowbase + lane, 1)], sm)
            return 0

        lax.fori_loop(0, _G, issue, 0)
        return rel_cps

    def drain(s):
        pltpu.make_async_copy(ent.at[pl.ds(0, _ER)], ebufs[s], sems[s]).wait()

    acc = jnp.zeros((_L,), jnp.float32)
    cps = fire(0, 0)
    for k in range(_NCH):
        nxt = fire(k + 1, (k + 1) % 2) if k + 1 < _NCH else None
        for cp in cps:
            cp.wait()
        drain(k % 2)
        eb, rb = ebufs[k % 2], rbufs[k % 2]

        def gbody(g, acc, eb=eb, rb=rb, k=k):
            rows = g * _L + iota
            gsl = pl.ds(k * _C + g * _L, _L)
            cb = [colb[j, gsl] for j in range(2)]

            def cbody(ci, rowsum, eb=eb, rb=rb, cb=cb, rows=rows):
                for u in range(_UNROLL):
                    c = ci * _UNROLL + u
                    col = jnp.full((_L,), c, jnp.int32)
                    a = plsc.load_gather(eb, [rows, col])
                    t = plsc.load_gather(eb, [_C + rows, col])
                    d = plsc.load_gather(eb, [2 * _C + rows, col])
                    e = plsc.load_gather(eb, [3 * _C + rows, col])
                    b = plsc.load_gather(rb.at[0], [rows, cb[0] + c])
                    f = plsc.load_gather(rb.at[1], [rows, cb[1] + c])
                    rowsum = rowsum + (jnp.abs(a + b - t) - jnp.abs(d + f - e))
                return rowsum

            rowsum = lax.fori_loop(0, _D // _UNROLL, cbody,
                                   jnp.zeros((_L,), jnp.float32))
            return acc + jnp.maximum(rowsum + _MARGIN, 0.0)

        acc = lax.fori_loop(0, _G, gbody, acc)
        cps = nxt

    obuf[...] = acc
    pltpu.sync_copy(obuf, out.at[wid])


def _transe_partials(xT, ent_emb, rel2):
    f32 = jnp.float32
    run = pl.kernel(
        _transe_body,
        mesh=plsc.VectorSubcoreMesh(core_axis_name="c", subcore_axis_name="s"),
        compiler_params=pltpu.CompilerParams(
            needs_layout_passes=False, use_tc_tiling_on_sc=True),
        out_type=jax.ShapeDtypeStruct((_NW, _L), f32),
        scratch_types=[
            pltpu.VMEM((6, _BPW), jnp.int32),
            pltpu.VMEM((2, _BPW), jnp.int32),
            pltpu.VMEM((2, _BPW), jnp.int32),
            pltpu.VMEM((_ER, _D), f32),
            pltpu.VMEM((_ER, _D), f32),
            pltpu.VMEM((2, _C, _W), f32),
            pltpu.VMEM((2, _C, _W), f32),
            pltpu.VMEM((_L,), f32),
            pltpu.SemaphoreType.DMA,
            pltpu.SemaphoreType.DMA,
            pltpu.SemaphoreType.DMA,
        ],
    )
    return run(xT, ent_emb, rel2)


def kernel(x, ent_emb, rel_emb):
    xT = x.T
    rel2 = rel_emb.reshape(500, _W)
    partials = _transe_partials(xT, ent_emb, rel2)
    return jnp.sum(partials)
